# baseline (device time: 84930 ns/iter reference)
import jax
import jax.numpy as jnp
from jax import lax
from jax.experimental import pallas as pl
from jax.experimental.pallas import tpu as pltpu

N_Z = 4
EPS = 1e-6


def kernel(partial, resid, gamma):
    m, d = resid.shape
    mq = m // 4
    mp = mq // N_Z

    def body(p_ref, r_ref, g_ref, out_ref,
             stage, comm_ref, allq, ostrip,
             ld_sem, sts_sems,
             rs_ssem, rs_rsem, zag_ssem, zag_rsem,
             xp_ssem, xp_rsem, yp_ssem, yp_rsem,
             xd_ssem, xd_rsem, yd_ssem, yd_rsem, dd_ssem, dd_rsem):
        my_x = lax.axis_index("x")
        my_y = lax.axis_index("y")
        my_z = lax.axis_index("z")
        right = (my_z + 1) % N_Z
        left = (my_z + N_Z - 1) % N_Z
        q = 2 * my_x + my_y
        qx = 2 * (1 - my_x) + my_y
        qy = 2 * my_x + (1 - my_y)
        qd = 2 * (1 - my_x) + (1 - my_y)
        x_nbr = (1 - my_x, my_y, my_z)
        y_nbr = (my_x, 1 - my_y, my_z)

        def piece_rows(quarter, j):
            return pl.ds(quarter * mq + j * mp, mp)

        def load_partial_piece(j):
            cp = pltpu.make_async_copy(
                p_ref.at[0, piece_rows(q, j), :], stage, ld_sem)
            cp.start()
            cp.wait()

        def piece_rdma(quarter, j, ssem, rsem, dev):
            rdma = pltpu.make_async_remote_copy(
                src_ref=allq.at[piece_rows(quarter, j), :],
                dst_ref=allq.at[piece_rows(quarter, j), :],
                send_sem=ssem, recv_sem=rsem,
                device_id=dev, device_id_type=pl.DeviceIdType.MESH,
            )
            rdma.start()
            return rdma

        load_partial_piece(my_z)
        comm_ref[0, :, :] = stage[:, :].astype(jnp.bfloat16)

        dd_to = (1 - my_x, 1 - my_y, (my_z + 3) % N_Z)
        dd_from = (1 - my_x, 1 - my_y, (my_z + 1) % N_Z)
        barrier_sem = pltpu.get_barrier_semaphore()
        for dev in [(my_x, my_y, left), (my_x, my_y, right), x_nbr, y_nbr,
                    dd_to, dd_from]:
            pl.semaphore_signal(
                barrier_sem, inc=1,
                device_id=dev, device_id_type=pl.DeviceIdType.MESH,
            )
        pl.semaphore_wait(barrier_sem, 6)

        for s in range(N_Z - 1):
            rdma = pltpu.make_async_remote_copy(
                src_ref=comm_ref.at[s % 2],
                dst_ref=comm_ref.at[(s + 1) % 2],
                send_sem=rs_ssem.at[s],
                recv_sem=rs_rsem.at[s],
                device_id=(my_x, my_y, right),
                device_id_type=pl.DeviceIdType.MESH,
            )
            rdma.start()
            load_partial_piece((my_z + (N_Z - 1) - s) % N_Z)
            rdma.wait()
            comm_ref[(s + 1) % 2, :, :] = (
                comm_ref[(s + 1) % 2, :, :] + stage[:, :].astype(jnp.bfloat16)
            )

        j_own = (my_z + 1) % N_Z
        cp = pltpu.make_async_copy(
            r_ref.at[piece_rows(q, j_own), :], stage, ld_sem)
        cp.start()
        cp.wait()
        y = comm_ref[1, :, :].astype(jnp.float32) + stage[:, :]
        rms = jnp.sqrt(jnp.mean(y * y, axis=-1, keepdims=True) + EPS)
        outc = (y / rms) * g_ref[0, :][None, :]
        allq[piece_rows(q, j_own), :] = outc.astype(jnp.bfloat16)

        sends = [
            piece_rdma(q, j_own, xp_ssem.at[0], xp_rsem.at[0], x_nbr),
            piece_rdma(q, j_own, yp_ssem.at[0], yp_rsem.at[0], y_nbr),
            piece_rdma(q, j_own, dd_ssem, dd_rsem, dd_to),
        ]

        for h in range(N_Z - 1):
            js = (my_z + 1 - h + N_Z) % N_Z
            jr = (my_z - h + N_Z) % N_Z
            rdma = pltpu.make_async_remote_copy(
                src_ref=allq.at[piece_rows(q, js), :],
                dst_ref=allq.at[piece_rows(q, js), :],
                send_sem=zag_ssem.at[h],
                recv_sem=zag_rsem.at[h],
                device_id=(my_x, my_y, right),
                device_id_type=pl.DeviceIdType.MESH,
            )
            rdma.start()
            rdma.wait()
            i = h + 1
            sends.append(piece_rdma(q, jr, xp_ssem.at[i], xp_rsem.at[i], x_nbr))
            sends.append(piece_rdma(q, jr, yp_ssem.at[i], yp_rsem.at[i], y_nbr))

        def strip_store(quarter, slot):
            ostrip[slot, :, :] = allq[pl.ds(quarter * mq, mq), :].astype(
                jnp.float32)
            sts = pltpu.make_async_copy(
                ostrip.at[slot], out_ref.at[pl.ds(quarter * mq, mq), :],
                sts_sems.at[slot])
            sts.start()
            return sts

        sts_pending = [strip_store(q, 0), None]

        def wait_recv(quarter, i, rsem):
            recv = pltpu.make_async_remote_copy(
                src_ref=allq.at[piece_rows(quarter, 0), :],
                dst_ref=allq.at[piece_rows(quarter, 0), :],
                send_sem=rsem, recv_sem=rsem,
                device_id=(my_x, my_y, my_z),
                device_id_type=pl.DeviceIdType.MESH,
            )
            recv.wait_recv()
            return (quarter * 0 + my_z + 1 - i + N_Z) % N_Z

        for i in range(2):
            j = wait_recv(qy, i, yp_rsem.at[i])
            sends.append(piece_rdma(qy, j, xd_ssem.at[i], xd_rsem.at[i], x_nbr))
        for i in range(3):
            j = wait_recv(qx, i, xp_rsem.at[i])
        sends.append(piece_rdma(qx, j, yd_ssem.at[0], yd_rsem.at[0], y_nbr))
        wait_recv(qx, 3, xp_rsem.at[3])
        sts_pending[1] = strip_store(qx, 1)
        for i in range(2, N_Z):
            wait_recv(qy, i, yp_rsem.at[i])
        sts_pending[0].wait()
        sts_pending[0] = strip_store(qy, 0)

        for i in range(2):
            wait_recv(qd, i, xd_rsem.at[i])
        wait_recv(qd, 0, yd_rsem.at[0])
        wait_recv(qd, 0, dd_rsem)
        sts_pending[1].wait()
        sts = strip_store(qd, 1)
        sts.wait()
        sts_pending[0].wait()
        for snd in sends:
            snd.wait_send()

    return pl.pallas_call(
        body,
        out_shape=jax.ShapeDtypeStruct((m, d), jnp.float32),
        in_specs=[
            pl.BlockSpec(memory_space=pl.ANY),
            pl.BlockSpec(memory_space=pl.ANY),
            pl.BlockSpec(memory_space=pltpu.VMEM),
        ],
        out_specs=pl.BlockSpec(memory_space=pl.ANY),
        scratch_shapes=[
            pltpu.VMEM((mp, d), jnp.float32),
            pltpu.VMEM((2, mp, d), jnp.bfloat16),
            pltpu.VMEM((m, d), jnp.bfloat16),
            pltpu.VMEM((2, mq, d), jnp.float32),
            pltpu.SemaphoreType.DMA,
            pltpu.SemaphoreType.DMA((2,)),
            pltpu.SemaphoreType.DMA((N_Z - 1,)),
            pltpu.SemaphoreType.DMA((N_Z - 1,)),
            pltpu.SemaphoreType.DMA((N_Z - 1,)),
            pltpu.SemaphoreType.DMA((N_Z - 1,)),
            pltpu.SemaphoreType.DMA((N_Z,)),
            pltpu.SemaphoreType.DMA((N_Z,)),
            pltpu.SemaphoreType.DMA((N_Z,)),
            pltpu.SemaphoreType.DMA((N_Z,)),
            pltpu.SemaphoreType.DMA((2,)),
            pltpu.SemaphoreType.DMA((2,)),
            pltpu.SemaphoreType.DMA((2,)),
            pltpu.SemaphoreType.DMA((2,)),
            pltpu.SemaphoreType.DMA,
            pltpu.SemaphoreType.DMA,
        ],
        compiler_params=pltpu.CompilerParams(collective_id=0),
    )(partial, resid, gamma.reshape(1, d))


# device time: 78680 ns/iter; 1.0794x vs baseline; 1.0794x over previous
import jax
import jax.numpy as jnp
from jax import lax
from jax.experimental import pallas as pl
from jax.experimental.pallas import tpu as pltpu

N_Z = 4
EPS = 1e-6


def kernel(partial, resid, gamma):
    m, d = resid.shape
    mq = m // 4
    mp = mq // N_Z

    def body(p_ref, r_ref, g_ref, out_ref,
             stage, comm_ref, allq, ostrip,
             ld_sem, sts_sems,
             rs_ssem, rs_rsem, zag_ssem, zag_rsem,
             xp_ssem, xp_rsem, yp_ssem, yp_rsem,
             xd_ssem, xd_rsem, yd_ssem, yd_rsem):
        my_x = lax.axis_index("x")
        my_y = lax.axis_index("y")
        my_z = lax.axis_index("z")
        right = (my_z + 1) % N_Z
        left = (my_z + N_Z - 1) % N_Z
        q = 2 * my_x + my_y
        qx = 2 * (1 - my_x) + my_y
        qy = 2 * my_x + (1 - my_y)
        qd = 2 * (1 - my_x) + (1 - my_y)
        x_nbr = (1 - my_x, my_y, my_z)
        y_nbr = (my_x, 1 - my_y, my_z)

        def piece_rows(quarter, j):
            return pl.ds(quarter * mq + j * mp, mp)

        def load_partial_piece(j):
            cp = pltpu.make_async_copy(
                p_ref.at[0, piece_rows(q, j), :], stage, ld_sem)
            cp.start()
            cp.wait()

        def piece_rdma(quarter, j, ssem, rsem, dev):
            rdma = pltpu.make_async_remote_copy(
                src_ref=allq.at[piece_rows(quarter, j), :],
                dst_ref=allq.at[piece_rows(quarter, j), :],
                send_sem=ssem, recv_sem=rsem,
                device_id=dev, device_id_type=pl.DeviceIdType.MESH,
            )
            rdma.start()
            return rdma

        load_partial_piece(my_z)
        comm_ref[0, :, :] = stage[:, :].astype(jnp.bfloat16)

        barrier_sem = pltpu.get_barrier_semaphore()
        for dev in [(my_x, my_y, left), (my_x, my_y, right), x_nbr, y_nbr]:
            pl.semaphore_signal(
                barrier_sem, inc=1,
                device_id=dev, device_id_type=pl.DeviceIdType.MESH,
            )
        pl.semaphore_wait(barrier_sem, 4)

        for s in range(N_Z - 1):
            rdma = pltpu.make_async_remote_copy(
                src_ref=comm_ref.at[s % 2],
                dst_ref=comm_ref.at[(s + 1) % 2],
                send_sem=rs_ssem.at[s],
                recv_sem=rs_rsem.at[s],
                device_id=(my_x, my_y, right),
                device_id_type=pl.DeviceIdType.MESH,
            )
            rdma.start()
            load_partial_piece((my_z + (N_Z - 1) - s) % N_Z)
            rdma.wait()
            comm_ref[(s + 1) % 2, :, :] = (
                comm_ref[(s + 1) % 2, :, :] + stage[:, :].astype(jnp.bfloat16)
            )

        j_own = (my_z + 1) % N_Z
        cp = pltpu.make_async_copy(
            r_ref.at[piece_rows(q, j_own), :], stage, ld_sem)
        cp.start()
        cp.wait()
        y = comm_ref[1, :, :].astype(jnp.float32) + stage[:, :]
        rms = jnp.sqrt(jnp.mean(y * y, axis=-1, keepdims=True) + EPS)
        outc = (y / rms) * g_ref[0, :][None, :]
        allq[piece_rows(q, j_own), :] = outc.astype(jnp.bfloat16)

        sends = [
            piece_rdma(q, j_own, xp_ssem.at[0], xp_rsem.at[0], x_nbr),
            piece_rdma(q, j_own, yp_ssem.at[0], yp_rsem.at[0], y_nbr),
        ]

        for h in range(N_Z - 1):
            js = (my_z + 1 - h + N_Z) % N_Z
            jr = (my_z - h + N_Z) % N_Z
            rdma = pltpu.make_async_remote_copy(
                src_ref=allq.at[piece_rows(q, js), :],
                dst_ref=allq.at[piece_rows(q, js), :],
                send_sem=zag_ssem.at[h],
                recv_sem=zag_rsem.at[h],
                device_id=(my_x, my_y, right),
                device_id_type=pl.DeviceIdType.MESH,
            )
            rdma.start()
            rdma.wait()
            i = h + 1
            sends.append(piece_rdma(q, jr, xp_ssem.at[i], xp_rsem.at[i], x_nbr))
            sends.append(piece_rdma(q, jr, yp_ssem.at[i], yp_rsem.at[i], y_nbr))

        def strip_store(quarter, slot):
            ostrip[slot, :, :] = allq[pl.ds(quarter * mq, mq), :].astype(
                jnp.float32)
            sts = pltpu.make_async_copy(
                ostrip.at[slot], out_ref.at[pl.ds(quarter * mq, mq), :],
                sts_sems.at[slot])
            sts.start()
            return sts

        sts_pending = [strip_store(q, 0), None]

        def wait_recv(quarter, i, rsem):
            recv = pltpu.make_async_remote_copy(
                src_ref=allq.at[piece_rows(quarter, 0), :],
                dst_ref=allq.at[piece_rows(quarter, 0), :],
                send_sem=rsem, recv_sem=rsem,
                device_id=(my_x, my_y, my_z),
                device_id_type=pl.DeviceIdType.MESH,
            )
            recv.wait_recv()
            return (quarter * 0 + my_z + 1 - i + N_Z) % N_Z

        for i in range(2):
            j = wait_recv(qy, i, yp_rsem.at[i])
            sends.append(piece_rdma(qy, j, xd_ssem.at[i], xd_rsem.at[i], x_nbr))
        for i in range(3):
            j = wait_recv(qx, i, xp_rsem.at[i])
        sends.append(piece_rdma(qx, j, yd_ssem.at[0], yd_rsem.at[0], y_nbr))
        j = wait_recv(qx, 3, xp_rsem.at[3])
        sends.append(piece_rdma(qx, j, yd_ssem.at[1], yd_rsem.at[1], y_nbr))
        sts_pending[1] = strip_store(qx, 1)
        for i in range(2, N_Z):
            wait_recv(qy, i, yp_rsem.at[i])
        sts_pending[0].wait()
        sts_pending[0] = strip_store(qy, 0)

        for i in range(2):
            wait_recv(qd, i, xd_rsem.at[i])
            wait_recv(qd, i, yd_rsem.at[i])
        sts_pending[1].wait()
        sts = strip_store(qd, 1)
        sts.wait()
        sts_pending[0].wait()
        for snd in sends:
            snd.wait_send()

    return pl.pallas_call(
        body,
        out_shape=jax.ShapeDtypeStruct((m, d), jnp.float32),
        in_specs=[
            pl.BlockSpec(memory_space=pl.ANY),
            pl.BlockSpec(memory_space=pl.ANY),
            pl.BlockSpec(memory_space=pltpu.VMEM),
        ],
        out_specs=pl.BlockSpec(memory_space=pl.ANY),
        scratch_shapes=[
            pltpu.VMEM((mp, d), jnp.float32),
            pltpu.VMEM((2, mp, d), jnp.bfloat16),
            pltpu.VMEM((m, d), jnp.bfloat16),
            pltpu.VMEM((2, mq, d), jnp.float32),
            pltpu.SemaphoreType.DMA,
            pltpu.SemaphoreType.DMA((2,)),
            pltpu.SemaphoreType.DMA((N_Z - 1,)),
            pltpu.SemaphoreType.DMA((N_Z - 1,)),
            pltpu.SemaphoreType.DMA((N_Z - 1,)),
            pltpu.SemaphoreType.DMA((N_Z - 1,)),
            pltpu.SemaphoreType.DMA((N_Z,)),
            pltpu.SemaphoreType.DMA((N_Z,)),
            pltpu.SemaphoreType.DMA((N_Z,)),
            pltpu.SemaphoreType.DMA((N_Z,)),
            pltpu.SemaphoreType.DMA((2,)),
            pltpu.SemaphoreType.DMA((2,)),
            pltpu.SemaphoreType.DMA((2,)),
            pltpu.SemaphoreType.DMA((2,)),
        ],
        compiler_params=pltpu.CompilerParams(collective_id=0),
    )(partial, resid, gamma.reshape(1, d))
